# Initial kernel scaffold; baseline (speedup 1.0000x reference)
import jax
import jax.numpy as jnp
from jax.experimental import pallas as pl


def kernel(msg, atom, bond, src, dst, W, b, gamma, beta):
    raise NotImplementedError("write your pallas kernel here")


# speedup vs baseline: 2.6955x; 2.6955x over previous
import functools

import jax
import jax.numpy as jnp
from jax import lax
from jax.experimental import pallas as pl
from jax.experimental.pallas import tpu as pltpu
from jax.experimental.pallas import tpu_sc as plsc

NN = 10000
NE = 320000
MD = 128
AD = 128
BD = 16

NC = 2
NS = 16
NW = NC * NS
EPW = NE // NW
CHUNK = 80
NITER = EPW // CHUNK


def _sc_mesh():
    return plsc.VectorSubcoreMesh(core_axis_name="c", subcore_axis_name="s")


def _scatter_add(msg, dst, zeros):
    @functools.partial(
        pl.kernel,
        mesh=_sc_mesh(),
        out_type=jax.ShapeDtypeStruct((NC * NN, MD), jnp.float32),
        scratch_types=[
            pltpu.VMEM((CHUNK,), jnp.int32),
            pltpu.VMEM((CHUNK, MD), jnp.float32),
            pltpu.VMEM_SHARED((NN, MD), jnp.float32),
        ],
    )
    def k(msg_hbm, dst_hbm, zeros_hbm, out_hbm, idx_v, rows_v, inc_sh):
        cid = lax.axis_index("c")
        sid = lax.axis_index("s")
        wid = sid * NC + cid
        rpt = 1000

        @pl.when(sid < NN // rpt)
        def _():
            pltpu.sync_copy(zeros_hbm.at[pl.ds(sid * rpt, rpt)],
                            inc_sh.at[pl.ds(sid * rpt, rpt)])

        plsc.subcore_barrier()

        def body(i, carry):
            base = wid * EPW + i * CHUNK
            pltpu.sync_copy(dst_hbm.at[pl.ds(base, CHUNK)], idx_v)
            pltpu.sync_copy(msg_hbm.at[pl.ds(base, CHUNK)], rows_v)
            pltpu.sync_copy(rows_v, inc_sh.at[idx_v], add=True)
            return carry

        lax.fori_loop(0, NITER, body, 0)
        plsc.subcore_barrier()

        @pl.when(sid < NN // rpt)
        def _():
            pltpu.sync_copy(inc_sh.at[pl.ds(sid * rpt, rpt)],
                            out_hbm.at[pl.ds(cid * NN + sid * rpt, rpt)])

    return k(msg, dst, zeros)


def _gather_rows(nf, src):
    @functools.partial(
        pl.kernel,
        mesh=_sc_mesh(),
        out_type=jax.ShapeDtypeStruct((NE, MD), jnp.float32),
        scratch_types=[
            pltpu.VMEM((CHUNK,), jnp.int32),
            pltpu.VMEM((CHUNK, MD), jnp.float32),
            pltpu.SemaphoreType.DMA,
        ],
    )
    def k(nf_hbm, src_hbm, out_hbm, idx_v, rows_v, sem):
        cid = lax.axis_index("c")
        sid = lax.axis_index("s")
        wid = sid * NC + cid

        def body(i, carry):
            base = wid * EPW + i * CHUNK
            pltpu.sync_copy(src_hbm.at[pl.ds(base, CHUNK)], idx_v)
            pltpu.async_copy(nf_hbm.at[idx_v], rows_v, sem).wait()
            pltpu.sync_copy(rows_v, out_hbm.at[pl.ds(base, CHUNK)])
            return carry

        lax.fori_loop(0, NITER, body, 0)

    return k(nf, src)


_NF_BLK = 2000


def _node_features(inc0, inc1, atom, WaT, WcT, b):
    def body(i0_ref, i1_ref, a_ref, wa_ref, wc_ref, b_ref, o_ref):
        inc = i0_ref[...] + i1_ref[...]
        o_ref[...] = (
            jnp.dot(a_ref[...], wa_ref[...], preferred_element_type=jnp.float32)
            + jnp.dot(inc, wc_ref[...], preferred_element_type=jnp.float32)
            + b_ref[...]
        )

    blk = pl.BlockSpec((_NF_BLK, MD), lambda i: (i, 0))
    full = pl.BlockSpec((MD, MD), lambda i: (0, 0))
    vec = pl.BlockSpec((1, MD), lambda i: (0, 0))
    return pl.pallas_call(
        body,
        grid=(NN // _NF_BLK,),
        in_specs=[blk, blk, blk, full, full, vec],
        out_specs=blk,
        out_shape=jax.ShapeDtypeStruct((NN, MD), jnp.float32),
    )(inc0, inc1, atom, WaT, WcT, b)


_ED_BLK = 2000


def _edge_epilogue(msg, gathered, bond, WbT, gamma, beta):
    def body(m_ref, g_ref, bo_ref, wb_ref, ga_ref, be_ref, o_ref):
        bp = jnp.dot(bo_ref[...], wb_ref[...], preferred_element_type=jnp.float32)
        upd = jnp.maximum(g_ref[...] + bp, 0.0)
        x = m_ref[...] + upd
        mu = jnp.mean(x, axis=1, keepdims=True)
        xc = x - mu
        var = jnp.mean(xc * xc, axis=1, keepdims=True)
        o_ref[...] = xc * lax.rsqrt(var + 1e-5) * ga_ref[...] + be_ref[...]

    blk = pl.BlockSpec((_ED_BLK, MD), lambda i: (i, 0))
    bblk = pl.BlockSpec((_ED_BLK, BD), lambda i: (i, 0))
    wblk = pl.BlockSpec((BD, MD), lambda i: (0, 0))
    vec = pl.BlockSpec((1, MD), lambda i: (0, 0))
    return pl.pallas_call(
        body,
        grid=(NE // _ED_BLK,),
        in_specs=[blk, blk, bblk, wblk, vec, vec],
        out_specs=blk,
        out_shape=jax.ShapeDtypeStruct((NE, MD), jnp.float32),
    )(msg, gathered, bond, WbT, gamma, beta)


def kernel(msg, atom, bond, src, dst, W, b, gamma, beta):
    src = src.astype(jnp.int32)
    dst = dst.astype(jnp.int32)
    WaT = W[:, :AD].T
    WbT = W[:, AD:AD + BD].T
    WcT = W[:, AD + BD:].T
    zeros = jnp.zeros((NN, MD), jnp.float32)

    inc2 = _scatter_add(msg, dst, zeros)
    nf = _node_features(inc2[:NN], inc2[NN:], atom, WaT, WcT,
                        b.reshape(1, MD))
    gathered = _gather_rows(nf, src)
    return _edge_epilogue(msg, gathered, bond, WbT,
                          gamma.reshape(1, MD), beta.reshape(1, MD))


# speedup vs baseline: 3.9461x; 1.4639x over previous
import functools

import jax
import jax.numpy as jnp
from jax import lax
from jax.experimental import pallas as pl
from jax.experimental.pallas import tpu as pltpu
from jax.experimental.pallas import tpu_sc as plsc

NN = 10000
NE = 320000
MD = 128
AD = 128
BD = 16

NC = 2
NS = 16
NW = NC * NS
EPW = NE // NW
CHUNK = 80
NITER = EPW // CHUNK


def _sc_mesh():
    return plsc.VectorSubcoreMesh(core_axis_name="c", subcore_axis_name="s")


_NBUF = 3


def _scatter_add(msg, dst3, zeros):
    @functools.partial(
        pl.kernel,
        mesh=_sc_mesh(),
        out_type=jax.ShapeDtypeStruct((NC * NN, MD), jnp.float32),
        scratch_types=[
            pltpu.VMEM((NITER, CHUNK), jnp.int32),
            *[pltpu.VMEM((CHUNK, MD), jnp.float32) for _ in range(_NBUF)],
            *[pltpu.SemaphoreType.DMA for _ in range(_NBUF)],
            pltpu.VMEM_SHARED((NN, MD), jnp.float32),
        ],
    )
    def k(msg_hbm, dst_hbm, zeros_hbm, out_hbm, idx_v, *rest):
        bufs = rest[:_NBUF]
        sems = rest[_NBUF:2 * _NBUF]
        inc_sh = rest[2 * _NBUF]
        cid = lax.axis_index("c")
        sid = lax.axis_index("s")
        wid = sid * NC + cid
        rpt = 1000

        @pl.when(sid < NN // rpt)
        def _():
            pltpu.sync_copy(zeros_hbm.at[pl.ds(sid * rpt, rpt)],
                            inc_sh.at[pl.ds(sid * rpt, rpt)])

        pltpu.sync_copy(dst_hbm.at[wid], idx_v)
        plsc.subcore_barrier()

        loads = [None] * _NBUF

        def start_load(b, i):
            base = wid * EPW + i * CHUNK
            loads[b] = pltpu.async_copy(
                msg_hbm.at[pl.ds(base, CHUNK)], bufs[b], sems[b])

        for b in range(_NBUF):
            start_load(b, b)
        for i in range(NITER):
            b = i % _NBUF
            loads[b].wait()
            pltpu.sync_copy(bufs[b], inc_sh.at[idx_v.at[i]], add=True)
            if i + _NBUF < NITER:
                start_load(b, i + _NBUF)

        plsc.subcore_barrier()

        @pl.when(sid < NN // rpt)
        def _():
            pltpu.sync_copy(inc_sh.at[pl.ds(sid * rpt, rpt)],
                            out_hbm.at[pl.ds(cid * NN + sid * rpt, rpt)])

    return k(msg, dst3, zeros)


_GRP = 5
_NG = NITER // _GRP
_GROWS = _GRP * CHUNK


def _gather_rows(nf, src3):
    @functools.partial(
        pl.kernel,
        mesh=_sc_mesh(),
        out_type=jax.ShapeDtypeStruct((NE, MD), jnp.float32),
        scratch_types=[
            pltpu.VMEM((NITER, CHUNK), jnp.int32),
            pltpu.VMEM((_GROWS, MD), jnp.float32),
            pltpu.VMEM((_GROWS, MD), jnp.float32),
            pltpu.SemaphoreType.DMA,
            pltpu.SemaphoreType.DMA,
            pltpu.SemaphoreType.DMA,
        ],
    )
    def k(nf_hbm, src_hbm, out_hbm, idx_v, rows0, rows1, gsem, w0, w1):
        cid = lax.axis_index("c")
        sid = lax.axis_index("s")
        wid = sid * NC + cid
        pltpu.sync_copy(src_hbm.at[wid], idx_v)

        rows = (rows0, rows1)
        wsem = (w0, w1)
        wh = [None, None]
        for g in range(_NG):
            r = g & 1
            if wh[r] is not None:
                wh[r].wait()
            ghs = []
            for b in range(_GRP):
                i = g * _GRP + b
                ghs.append(pltpu.async_copy(
                    nf_hbm.at[idx_v.at[i]],
                    rows[r].at[pl.ds(b * CHUNK, CHUNK)], gsem))
            for h in ghs:
                h.wait()
            base = wid * EPW + g * _GROWS
            wh[r] = pltpu.async_copy(
                rows[r], out_hbm.at[pl.ds(base, _GROWS)], wsem[r])
        for h in wh:
            h.wait()

    return k(nf, src3)


_NF_BLK = 2000


def _node_features(inc0, inc1, atom, WaT, WcT, b):
    def body(i0_ref, i1_ref, a_ref, wa_ref, wc_ref, b_ref, o_ref):
        inc = i0_ref[...] + i1_ref[...]
        o_ref[...] = (
            jnp.dot(a_ref[...], wa_ref[...], preferred_element_type=jnp.float32)
            + jnp.dot(inc, wc_ref[...], preferred_element_type=jnp.float32)
            + b_ref[...]
        )

    blk = pl.BlockSpec((_NF_BLK, MD), lambda i: (i, 0))
    full = pl.BlockSpec((MD, MD), lambda i: (0, 0))
    vec = pl.BlockSpec((1, MD), lambda i: (0, 0))
    return pl.pallas_call(
        body,
        grid=(NN // _NF_BLK,),
        in_specs=[blk, blk, blk, full, full, vec],
        out_specs=blk,
        out_shape=jax.ShapeDtypeStruct((NN, MD), jnp.float32),
    )(inc0, inc1, atom, WaT, WcT, b)


_ED_BLK = 2000


def _edge_epilogue(msg, gathered, bond, WbT, gamma, beta):
    def body(m_ref, g_ref, bo_ref, wb_ref, ga_ref, be_ref, o_ref):
        bp = jnp.dot(bo_ref[...], wb_ref[...], preferred_element_type=jnp.float32)
        upd = jnp.maximum(g_ref[...] + bp, 0.0)
        x = m_ref[...] + upd
        mu = jnp.mean(x, axis=1, keepdims=True)
        xc = x - mu
        var = jnp.mean(xc * xc, axis=1, keepdims=True)
        o_ref[...] = xc * lax.rsqrt(var + 1e-5) * ga_ref[...] + be_ref[...]

    blk = pl.BlockSpec((_ED_BLK, MD), lambda i: (i, 0))
    bblk = pl.BlockSpec((_ED_BLK, BD), lambda i: (i, 0))
    wblk = pl.BlockSpec((BD, MD), lambda i: (0, 0))
    vec = pl.BlockSpec((1, MD), lambda i: (0, 0))
    return pl.pallas_call(
        body,
        grid=(NE // _ED_BLK,),
        in_specs=[blk, blk, bblk, wblk, vec, vec],
        out_specs=blk,
        out_shape=jax.ShapeDtypeStruct((NE, MD), jnp.float32),
    )(msg, gathered, bond, WbT, gamma, beta)


def kernel(msg, atom, bond, src, dst, W, b, gamma, beta):
    src3 = src.astype(jnp.int32).reshape(NW, NITER, CHUNK)
    dst3 = dst.astype(jnp.int32).reshape(NW, NITER, CHUNK)
    WaT = W[:, :AD].T
    WbT = W[:, AD:AD + BD].T
    WcT = W[:, AD + BD:].T
    zeros = jnp.zeros((NN, MD), jnp.float32)

    inc2 = _scatter_add(msg, dst3, zeros)
    nf = _node_features(inc2[:NN], inc2[NN:], atom, WaT, WcT,
                        b.reshape(1, MD))
    gathered = _gather_rows(nf, src3)
    return _edge_epilogue(msg, gathered, bond, WbT,
                          gamma.reshape(1, MD), beta.reshape(1, MD))
